# Initial kernel scaffold; baseline (speedup 1.0000x reference)
import jax
import jax.numpy as jnp
from jax.experimental import pallas as pl


def kernel(mem, transition, insert_position, sample_idx):
    raise NotImplementedError("write your pallas kernel here")


# speedup vs baseline: 5.1946x; 5.1946x over previous
import functools

import jax
import jax.numpy as jnp
from jax import lax
from jax.experimental import pallas as pl
from jax.experimental.pallas import tpu as pltpu
from jax.experimental.pallas import tpu_sc as plsc

MAX_SIZE = 1000000
D = 64
SAMPLE_BS = 4096
LANES = 16
PAD_ROWS = 8


def _make_sc_kernel(data_len: int, num_workers: int):
    chunk = SAMPLE_BS // num_workers
    n_vec = chunk // LANES
    mesh = plsc.VectorSubcoreMesh(core_axis_name="c", subcore_axis_name="s")
    num_cores = mesh.num_cores

    @functools.partial(
        pl.kernel,
        mesh=mesh,
        out_type=jax.ShapeDtypeStruct((SAMPLE_BS + PAD_ROWS, D), jnp.float32),
        compiler_params=pltpu.CompilerParams(use_tc_tiling_on_sc=False),
        scratch_types=[
            pltpu.VMEM((chunk,), jnp.int32),
            pltpu.VMEM((LANES,), jnp.int32),
            pltpu.VMEM((chunk,), jnp.int32),
            pltpu.VMEM((chunk,), jnp.int32),
            pltpu.VMEM((chunk,), jnp.int32),
            pltpu.VMEM((chunk, D), jnp.float32),
            pltpu.VMEM((chunk, D), jnp.float32),
            pltpu.SemaphoreType.DMA,
            pltpu.SemaphoreType.DMA,
        ],
    )
    def sc_kernel(mem_hbm, flat_hbm, ip_hbm, sidx_hbm, out_hbm,
                  idx_v, ip_v, fidx_v, pmem_v, pflat_v, mrows_v, frows_v,
                  sem0, sem1):
        wid = lax.axis_index("s") * num_cores + lax.axis_index("c")
        base = wid * chunk

        pltpu.sync_copy(sidx_hbm.at[pl.ds(base, chunk)], idx_v)
        pltpu.sync_copy(ip_hbm, ip_v)
        ip = ip_v[...]

        for c in range(n_vec):
            sl = pl.ds(c * LANES, LANES)
            iv = idx_v[sl]
            d = iv - ip
            rel = d + jnp.where(d < 0, MAX_SIZE, 0)
            in_win = rel < data_len
            offs = base + c * LANES + lax.iota(jnp.int32, LANES)
            fidx_v[sl] = jnp.where(in_win, rel, 0)
            pmem_v[sl] = jnp.where(in_win, SAMPLE_BS, offs)
            pflat_v[sl] = jnp.where(in_win, offs, SAMPLE_BS)

        g0 = pltpu.async_copy(mem_hbm.at[idx_v], mrows_v, sem0)
        g1 = pltpu.async_copy(flat_hbm.at[fidx_v], frows_v, sem1)
        g0.wait()
        g1.wait()
        s0 = pltpu.async_copy(mrows_v, out_hbm.at[pmem_v], sem0)
        s1 = pltpu.async_copy(frows_v, out_hbm.at[pflat_v], sem1)
        s0.wait()
        s1.wait()

    return sc_kernel


def kernel(mem, transition, insert_position, sample_idx):
    data_len = transition.shape[0] * transition.shape[1]
    flat = transition.reshape(data_len, transition.shape[2])
    info = plsc.get_sparse_core_info()
    num_workers = info.num_cores * info.num_subcores
    ip_arr = jnp.full((LANES,), insert_position, dtype=jnp.int32)
    sc = _make_sc_kernel(data_len, num_workers)
    out = sc(mem, flat, ip_arr, sample_idx.astype(jnp.int32))
    return out[:SAMPLE_BS]


# speedup vs baseline: 10.6977x; 2.0594x over previous
import functools

import jax
import jax.numpy as jnp
from jax import lax
from jax.experimental import pallas as pl
from jax.experimental.pallas import tpu as pltpu
from jax.experimental.pallas import tpu_sc as plsc

MAX_SIZE = 1000000
D = 64
SAMPLE_BS = 4096
LANES = 16


def _make_sc_kernel(data_len: int, num_workers: int):
    chunk = SAMPLE_BS // num_workers
    half = chunk // 2
    mesh = plsc.VectorSubcoreMesh(core_axis_name="c", subcore_axis_name="s")
    num_cores = mesh.num_cores

    @functools.partial(
        pl.kernel,
        mesh=mesh,
        out_type=jax.ShapeDtypeStruct((SAMPLE_BS, D), jnp.float32),
        compiler_params=pltpu.CompilerParams(needs_layout_passes=False),
        scratch_types=[
            pltpu.VMEM((chunk,), jnp.int32),
            pltpu.VMEM((chunk,), jnp.int32),
            pltpu.VMEM((chunk,), jnp.int32),
            pltpu.VMEM((LANES,), jnp.int32),
            pltpu.VMEM((half, 8, D), jnp.float32),
            pltpu.VMEM((chunk, D), jnp.float32),
            pltpu.SemaphoreType.DMA,
        ],
    )
    def sc_kernel(mem_hbm, flat_hbm, ip_hbm, sidx_hbm, out_hbm,
                  msk_v, blk_v, sub_v, ip_v, slabs_v, orows_v, sem0):
        wid = lax.axis_index("s") * num_cores + lax.axis_index("c")
        base = pl.multiple_of(wid * chunk, chunk)

        pltpu.sync_copy(sidx_hbm.at[pl.ds(base, chunk)], msk_v)
        pltpu.sync_copy(ip_hbm, ip_v)
        ip = ip_v[...]

        for c in range(chunk // LANES):
            sl = pl.ds(c * LANES, LANES)
            iv = msk_v[sl]
            d = iv - ip
            rel = d + jnp.where(d < 0, MAX_SIZE, 0)
            in_win = rel < data_len
            srow = jnp.where(in_win, rel, iv)
            blk_v[sl] = (srow >> 3) << 3
            sub_v[sl] = srow & 7
            msk_v[sl] = jnp.where(in_win, 1, 0)

        for b in range(2):
            for c16 in range(half // LANES):
                sl = pl.ds(b * half + c16 * LANES, LANES)
                bv = blk_v[sl]
                mv = msk_v[sl]
                for lane in range(LANES):
                    i = c16 * LANES + lane
                    blk = pl.multiple_of(bv[lane], 8)
                    m = mv[lane]

                    @pl.when(m == 1)
                    def _(blk=blk, i=i):
                        pltpu.async_copy(flat_hbm.at[pl.ds(blk, 8)],
                                         slabs_v.at[i], sem0)

                    @pl.when(m == 0)
                    def _(blk=blk, i=i):
                        pltpu.async_copy(mem_hbm.at[pl.ds(blk, 8)],
                                         slabs_v.at[i], sem0)

            for i in range(half):
                pltpu.make_async_copy(mem_hbm.at[pl.ds(0, 8)],
                                      slabs_v.at[i], sem0).wait()

            for c16 in range(half // LANES):
                sl = pl.ds(b * half + c16 * LANES, LANES)
                sv = sub_v[sl]
                for lane in range(LANES):
                    i = c16 * LANES + lane
                    s_sub = sv[lane]
                    for k in range(D // LANES):
                        ksl = pl.ds(k * LANES, LANES)
                        orows_v[b * half + i, ksl] = slabs_v[i, s_sub, ksl]

        pltpu.sync_copy(orows_v, out_hbm.at[pl.ds(base, chunk)])

    return sc_kernel


def kernel(mem, transition, insert_position, sample_idx):
    data_len = transition.shape[0] * transition.shape[1]
    flat = transition.reshape(data_len, transition.shape[2])
    info = plsc.get_sparse_core_info()
    num_workers = info.num_cores * info.num_subcores
    ip_arr = jnp.full((LANES,), insert_position, dtype=jnp.int32)
    sc = _make_sc_kernel(data_len, num_workers)
    return sc(mem, flat, ip_arr, sample_idx.astype(jnp.int32))
